# baseline (speedup 1.0000x reference)
import functools
import jax
import jax.numpy as jnp
from jax import lax
from jax.experimental import pallas as pl
from jax.experimental.pallas import tpu as pltpu, tpu_sc as plsc

NC = 2
NS = 16
NW = NC * NS

NODES = 1000000
B = 4096
L = 50
H = 32
TOTAL = B * L
B_PER_W = TOTAL // NW
CHUNK = 128
NCHUNK = B_PER_W // CHUNK

NB = 8
LOOKAHEAD = 4

W = 512
NBLK = NODES // W
TAIL = NODES - NBLK * W
BPT = -(-NBLK // NW)
HP = H + 1

_mesh = plsc.VectorSubcoreMesh(core_axis_name="c", subcore_axis_name="s")


@functools.partial(
    pl.kernel,
    out_type=jax.ShapeDtypeStruct((NODES * HP,), jnp.float32),
    mesh=_mesh,
    scratch_types=[
        pltpu.VMEM((2 * H, W), jnp.float32),
        pltpu.VMEM((2 * W * HP,), jnp.float32),
        pltpu.SemaphoreType.DMA((2,)),
        pltpu.SemaphoreType.DMA((2,)),
    ],
    compiler_params=pltpu.CompilerParams(
        use_tc_tiling_on_sc=True, needs_layout_passes=False
    ),
)
def _transpose_kernel(tab_t_hbm, tail_hbm, flat_hbm, stage_v, tbuf_v,
                      isems, osems):
    wid = lax.axis_index("s") * NC + lax.axis_index("c")
    start = wid * BPT
    stop = jnp.minimum(start + BPT, NBLK)

    iota_hp = jax.lax.iota(jnp.int32, 16) * HP

    def in_start(blk, b):
        pltpu.async_copy(tab_t_hbm.at[:, pl.ds(blk * W, W)],
                         stage_v.at[pl.ds(b * H, H)], isems.at[b])

    def in_wait(blk, b):
        pltpu.make_async_copy(tab_t_hbm.at[:, pl.ds(blk * W, W)],
                              stage_v.at[pl.ds(b * H, H)], isems.at[b]).wait()

    def out_start(blk, b):
        pltpu.async_copy(tbuf_v.at[pl.ds(b * W * HP, W * HP)],
                         flat_hbm.at[pl.ds(blk * W * HP, W * HP)], osems.at[b])

    def out_wait(blk, b):
        pltpu.make_async_copy(tbuf_v.at[pl.ds(b * W * HP, W * HP)],
                              flat_hbm.at[pl.ds(blk * W * HP, W * HP)],
                              osems.at[b]).wait()

    def transpose_block(b):
        @plsc.parallel_loop(0, W // 16, unroll=2)
        def _(g):
            rowvec = iota_hp + (b * W * HP + g * 16 * HP)
            for h in range(H):
                v = stage_v[b * H + h, pl.ds(g * 16, 16)]
                plsc.store_scatter(tbuf_v, [rowvec + h], v)

    in_start(start, 0)

    def body(i, carry):
        @pl.when(i % 2 == 0)
        def _():
            run(i, 0)

        @pl.when(i % 2 == 1)
        def _():
            run(i, 1)

        return carry

    def run(i, b):
        in_wait(i, b)

        @pl.when(i + 1 < stop)
        def _():
            in_start(i + 1, 1 - b)

        @pl.when(i >= start + 2)
        def _():
            out_wait(i - 2, b)

        transpose_block(b)
        out_start(i, b)

    lax.fori_loop(start, stop, body, 0)

    @pl.when(stop - start >= 2)
    def _():
        out_wait(stop - 2, (stop - 2) % 2)

    @pl.when(stop - start >= 1)
    def _():
        out_wait(stop - 1, (stop - 1) % 2)

    @pl.when(wid == NW - 1)
    def _():
        pltpu.sync_copy(tail_hbm, tbuf_v.at[pl.ds(0, TAIL * H)])
        for j in range(TAIL):
            v0 = tbuf_v[pl.ds(j * H, 16)]
            v1 = tbuf_v[pl.ds(j * H + 16, 16)]
            tbuf_v[pl.ds(W * HP + j * HP, 16)] = v0
            tbuf_v[pl.ds(W * HP + j * HP + 16, 16)] = v1
        pltpu.sync_copy(tbuf_v.at[pl.ds(W * HP, TAIL * HP)],
                        flat_hbm.at[pl.ds(NBLK * W * HP, TAIL * HP)])


@functools.partial(
    pl.kernel,
    out_type=jax.ShapeDtypeStruct((TOTAL, H), jnp.float32),
    mesh=_mesh,
    scratch_types=[
        pltpu.VMEM((NCHUNK, CHUNK), jnp.int32),
        pltpu.VMEM((NB, CHUNK, HP), jnp.float32),
        pltpu.SemaphoreType.DMA((NB,)),
        pltpu.SemaphoreType.DMA((NB,)),
    ],
    compiler_params=pltpu.CompilerParams(use_tc_tiling_on_sc=False),
)
def _gather_kernel(idx_hbm, table_hbm, out_hbm, idx_v, rows_v, gsems, osems):
    wid = lax.axis_index("s") * NC + lax.axis_index("c")
    base = wid * B_PER_W
    pltpu.sync_copy(idx_hbm.at[wid], idx_v)

    def gather_start(j, b):
        pltpu.async_copy(table_hbm.at[idx_v.at[j]], rows_v.at[b], gsems.at[b])

    def gather_wait(j, b):
        pltpu.make_async_copy(
            table_hbm.at[idx_v.at[j]], rows_v.at[b], gsems.at[b]
        ).wait()

    def out_start(j, b):
        pltpu.async_copy(
            rows_v.at[b, :, pl.ds(0, H)],
            out_hbm.at[pl.ds(base + j * CHUNK, CHUNK)], osems.at[b]
        )

    def out_wait(j, b):
        pltpu.make_async_copy(
            rows_v.at[b, :, pl.ds(0, H)],
            out_hbm.at[pl.ds(base + j * CHUNK, CHUNK)], osems.at[b]
        ).wait()

    for b in range(LOOKAHEAD):
        gather_start(b, b)

    def body(j, carry):
        b = j % NB
        gather_wait(j, b)
        out_start(j, b)
        jn = j + LOOKAHEAD
        bn = jn % NB

        @pl.when(jn < NCHUNK)
        def _():
            @pl.when(jn >= NB)
            def _():
                out_wait(jn - NB, bn)

            gather_start(jn, bn)

        return carry

    lax.fori_loop(0, NCHUNK, body, 0)

    for t in range(NCHUNK - NB, NCHUNK):
        out_wait(t, t % NB)


def kernel(conditions, table):
    tail = table[NBLK * W:].reshape(TAIL * H)
    flat = _transpose_kernel(table.T, tail)
    t_lin = flat.reshape(NODES, HP)
    idx = conditions.reshape(NW, NCHUNK, CHUNK)
    out = _gather_kernel(idx, t_lin)
    return out.reshape(B, L * H)


# speedup vs baseline: 7.5244x; 7.5244x over previous
import functools
import jax
import jax.numpy as jnp
from jax import lax
from jax.experimental import pallas as pl
from jax.experimental.pallas import tpu as pltpu, tpu_sc as plsc

NC = 2
NS = 16
NW = NC * NS

NODES = 1000000
B = 4096
L = 50
H = 32
TOTAL = B * L
B_PER_W = TOTAL // NW
CHUNK = 128
NCHUNK = B_PER_W // CHUNK

NB = 8
LOOKAHEAD = 4

W = 512
NBLK = NODES // W
TAIL = NODES - NBLK * W
BPT = -(-NBLK // NW)
HP = H + 1

_mesh = plsc.VectorSubcoreMesh(core_axis_name="c", subcore_axis_name="s")


@functools.partial(
    pl.kernel,
    out_type=jax.ShapeDtypeStruct((NODES * H,), jnp.float32),
    mesh=_mesh,
    scratch_types=[
        pltpu.VMEM((2 * H, W), jnp.float32),
        pltpu.VMEM((2 * W * HP,), jnp.float32),
        pltpu.VMEM((2 * W * H,), jnp.float32),
        pltpu.SemaphoreType.DMA((2,)),
        pltpu.SemaphoreType.DMA((2,)),
    ],
    compiler_params=pltpu.CompilerParams(
        use_tc_tiling_on_sc=True, needs_layout_passes=False
    ),
)
def _transpose_kernel(tab_t_hbm, tail_hbm, flat_hbm, stage_v, tbuf_v, cbuf_v,
                      isems, osems):
    wid = lax.axis_index("s") * NC + lax.axis_index("c")
    start = wid * BPT
    stop = jnp.minimum(start + BPT, NBLK)

    iota_hp = jax.lax.iota(jnp.int32, 16) * HP

    def in_start(blk, b):
        pltpu.async_copy(tab_t_hbm.at[:, pl.ds(blk * W, W)],
                         stage_v.at[pl.ds(b * H, H)], isems.at[b])

    def in_wait(blk, b):
        pltpu.make_async_copy(tab_t_hbm.at[:, pl.ds(blk * W, W)],
                              stage_v.at[pl.ds(b * H, H)], isems.at[b]).wait()

    def out_start(blk, b):
        pltpu.async_copy(cbuf_v.at[pl.ds(b * W * H, W * H)],
                         flat_hbm.at[pl.ds(blk * W * H, W * H)], osems.at[b])

    def out_wait(blk, b):
        pltpu.make_async_copy(cbuf_v.at[pl.ds(b * W * H, W * H)],
                              flat_hbm.at[pl.ds(blk * W * H, W * H)],
                              osems.at[b]).wait()

    def transpose_block(b):
        @plsc.parallel_loop(0, W // 16, unroll=2)
        def _(g):
            rowvec = iota_hp + (b * W * HP + g * 16 * HP)
            for h in range(H):
                v = stage_v[b * H + h, pl.ds(g * 16, 16)]
                plsc.store_scatter(tbuf_v, [rowvec + h], v)

        @plsc.parallel_loop(0, W, unroll=4)
        def _(j):
            src = b * W * HP + j * HP
            dst = b * W * H + j * H
            cbuf_v[pl.ds(dst, 16)] = tbuf_v[pl.ds(src, 16)]
            cbuf_v[pl.ds(dst + 16, 16)] = tbuf_v[pl.ds(src + 16, 16)]

    in_start(start, 0)

    def body(i, carry):
        @pl.when(i % 2 == 0)
        def _():
            run(i, 0)

        @pl.when(i % 2 == 1)
        def _():
            run(i, 1)

        return carry

    def run(i, b):
        in_wait(i, b)

        @pl.when(i + 1 < stop)
        def _():
            in_start(i + 1, 1 - b)

        @pl.when(i >= start + 2)
        def _():
            out_wait(i - 2, b)

        transpose_block(b)
        out_start(i, b)

    lax.fori_loop(start, stop, body, 0)

    @pl.when(stop - start >= 2)
    def _():
        out_wait(stop - 2, (stop - 2) % 2)

    @pl.when(stop - start >= 1)
    def _():
        out_wait(stop - 1, (stop - 1) % 2)

    @pl.when(wid == NW - 1)
    def _():
        pltpu.sync_copy(tail_hbm, cbuf_v.at[pl.ds(0, TAIL * H)])
        pltpu.sync_copy(cbuf_v.at[pl.ds(0, TAIL * H)],
                        flat_hbm.at[pl.ds(NBLK * W * H, TAIL * H)])


@functools.partial(
    pl.kernel,
    out_type=jax.ShapeDtypeStruct((TOTAL, H), jnp.float32),
    mesh=_mesh,
    scratch_types=[
        pltpu.VMEM((NCHUNK, CHUNK), jnp.int32),
        pltpu.VMEM((NB, CHUNK, H), jnp.float32),
        pltpu.SemaphoreType.DMA((NB,)),
        pltpu.SemaphoreType.DMA((NB,)),
    ],
    compiler_params=pltpu.CompilerParams(use_tc_tiling_on_sc=False),
)
def _gather_kernel(idx_hbm, table_hbm, out_hbm, idx_v, rows_v, gsems, osems):
    wid = lax.axis_index("s") * NC + lax.axis_index("c")
    base = wid * B_PER_W
    pltpu.sync_copy(idx_hbm.at[wid], idx_v)

    def gather_start(j, b):
        pltpu.async_copy(table_hbm.at[idx_v.at[j]], rows_v.at[b], gsems.at[b])

    def gather_wait(j, b):
        pltpu.make_async_copy(
            table_hbm.at[idx_v.at[j]], rows_v.at[b], gsems.at[b]
        ).wait()

    def out_start(j, b):
        pltpu.async_copy(
            rows_v.at[b], out_hbm.at[pl.ds(base + j * CHUNK, CHUNK)], osems.at[b]
        )

    def out_wait(j, b):
        pltpu.make_async_copy(
            rows_v.at[b], out_hbm.at[pl.ds(base + j * CHUNK, CHUNK)], osems.at[b]
        ).wait()

    for b in range(LOOKAHEAD):
        gather_start(b, b)

    def body(j, carry):
        b = j % NB
        gather_wait(j, b)
        out_start(j, b)
        jn = j + LOOKAHEAD
        bn = jn % NB

        @pl.when(jn < NCHUNK)
        def _():
            @pl.when(jn >= NB)
            def _():
                out_wait(jn - NB, bn)

            gather_start(jn, bn)

        return carry

    lax.fori_loop(0, NCHUNK, body, 0)

    for t in range(NCHUNK - NB, NCHUNK):
        out_wait(t, t % NB)


def kernel(conditions, table):
    tail = table[NBLK * W:].reshape(TAIL * H)
    flat = _transpose_kernel(table.T, tail)
    t_lin = flat.reshape(NODES, H)
    idx = conditions.reshape(NW, NCHUNK, CHUNK)
    out = _gather_kernel(idx, t_lin)
    return out.reshape(B, L * H)


# speedup vs baseline: 9.0905x; 1.2081x over previous
import functools
import jax
import jax.numpy as jnp
from jax import lax
from jax.experimental import pallas as pl
from jax.experimental.pallas import tpu as pltpu, tpu_sc as plsc

NC = 2
NS = 16
NW = NC * NS

NODES = 1000000
B = 4096
L = 50
H = 32
TOTAL = B * L
B_PER_W = TOTAL // NW
CHUNK = 128
NCHUNK = B_PER_W // CHUNK

NB = 8
LOOKAHEAD = 4

W = 512
NBLK = NODES // W
TAIL = NODES - NBLK * W
BPT = -(-NBLK // NW)
HP = H + 1

_mesh = plsc.VectorSubcoreMesh(core_axis_name="c", subcore_axis_name="s")


@functools.partial(
    pl.kernel,
    out_type=jax.ShapeDtypeStruct((NODES * H,), jnp.float32),
    mesh=_mesh,
    scratch_types=[
        pltpu.VMEM((2 * H, W), jnp.float32),
        pltpu.VMEM((2 * W * HP,), jnp.float32),
        pltpu.VMEM((2 * W * H,), jnp.float32),
        pltpu.SemaphoreType.DMA((2,)),
        pltpu.SemaphoreType.DMA((2,)),
    ],
    compiler_params=pltpu.CompilerParams(
        use_tc_tiling_on_sc=True, needs_layout_passes=False
    ),
)
def _transpose_kernel(tab_t_hbm, tail_hbm, flat_hbm, stage_v, tbuf_v, cbuf_v,
                      isems, osems):
    wid = lax.axis_index("s") * NC + lax.axis_index("c")
    start = wid * BPT
    stop = jnp.minimum(start + BPT, NBLK)

    iota_hp = jax.lax.iota(jnp.int32, 16) * HP

    def in_start(blk, b):
        pltpu.async_copy(tab_t_hbm.at[:, pl.ds(blk * W, W)],
                         stage_v.at[pl.ds(b * H, H)], isems.at[b])

    def in_wait(blk, b):
        pltpu.make_async_copy(tab_t_hbm.at[:, pl.ds(blk * W, W)],
                              stage_v.at[pl.ds(b * H, H)], isems.at[b]).wait()

    def out_start(blk, b):
        pltpu.async_copy(cbuf_v.at[pl.ds(b * W * H, W * H)],
                         flat_hbm.at[pl.ds(blk * W * H, W * H)], osems.at[b])

    def out_wait(blk, b):
        pltpu.make_async_copy(cbuf_v.at[pl.ds(b * W * H, W * H)],
                              flat_hbm.at[pl.ds(blk * W * H, W * H)],
                              osems.at[b]).wait()

    def transpose_block(b):
        @plsc.parallel_loop(0, W // 16, unroll=2)
        def _(g):
            rowvec = iota_hp + (b * W * HP + g * 16 * HP)
            for h in range(H):
                v = stage_v[b * H + h, pl.ds(g * 16, 16)]
                plsc.store_scatter(tbuf_v, [rowvec + h], v)

        @plsc.parallel_loop(0, W, unroll=4)
        def _(j):
            src = b * W * HP + j * HP
            dst = b * W * H + j * H
            cbuf_v[pl.ds(dst, 16)] = tbuf_v[pl.ds(src, 16)]
            cbuf_v[pl.ds(dst + 16, 16)] = tbuf_v[pl.ds(src + 16, 16)]

    in_start(start, 0)

    def body(i, carry):
        @pl.when(i % 2 == 0)
        def _():
            run(i, 0)

        @pl.when(i % 2 == 1)
        def _():
            run(i, 1)

        return carry

    def run(i, b):
        in_wait(i, b)

        @pl.when(i + 1 < stop)
        def _():
            in_start(i + 1, 1 - b)

        @pl.when(i >= start + 2)
        def _():
            out_wait(i - 2, b)

        transpose_block(b)
        out_start(i, b)

    lax.fori_loop(start, stop, body, 0)

    @pl.when(stop - start >= 2)
    def _():
        out_wait(stop - 2, (stop - 2) % 2)

    @pl.when(stop - start >= 1)
    def _():
        out_wait(stop - 1, (stop - 1) % 2)

    @pl.when(wid == NW - 1)
    def _():
        pltpu.sync_copy(tail_hbm, cbuf_v.at[pl.ds(0, TAIL * H)])
        pltpu.sync_copy(cbuf_v.at[pl.ds(0, TAIL * H)],
                        flat_hbm.at[pl.ds(NBLK * W * H, TAIL * H)])


@functools.partial(
    pl.kernel,
    out_type=jax.ShapeDtypeStruct((L * 4, NW, 8 * CHUNK), jnp.float32),
    mesh=_mesh,
    scratch_types=[
        pltpu.VMEM((NCHUNK, CHUNK), jnp.int32),
        pltpu.VMEM((NB, CHUNK, H), jnp.float32),
        pltpu.VMEM((2 * 4 * 8 * CHUNK,), jnp.float32),
        pltpu.SemaphoreType.DMA((NB,)),
        pltpu.SemaphoreType.DMA((2,)),
    ],
    compiler_params=pltpu.CompilerParams(
        use_tc_tiling_on_sc=False, needs_layout_passes=False
    ),
)
def _gather_kernel(idx_hbm, table_hbm, out_hbm, idx_v, rows_v, obuf_v,
                   gsems, osems):
    wid = lax.axis_index("s") * NC + lax.axis_index("c")
    pltpu.sync_copy(idx_hbm.at[wid], idx_v)

    iota16 = jax.lax.iota(jnp.int32, 16)

    def gather_start(j, b):
        pltpu.async_copy(table_hbm.at[idx_v.at[j]], rows_v.at[b], gsems.at[b])

    def gather_wait(j, b):
        pltpu.make_async_copy(table_hbm.at[idx_v.at[j]], rows_v.at[b],
                              gsems.at[b]).wait()

    def out_start(l, ob):
        for i in range(4):
            pltpu.async_copy(obuf_v.at[pl.ds(ob * 4096 + i * 1024, 1024)],
                             out_hbm.at[4 * l + i, wid], osems.at[ob])

    def out_wait(l, ob):
        for i in range(4):
            pltpu.make_async_copy(obuf_v.at[pl.ds(ob * 4096 + i * 1024, 1024)],
                                  out_hbm.at[4 * l + i, wid],
                                  osems.at[ob]).wait()

    def transpose_chunk(b, ob):
        for h in range(H):
            hdiag = (h + iota16) & 31
            sbase = ((hdiag >> 3) << 10) + ((hdiag & 7) << 7) + iota16 \
                + ob * 4096

            @plsc.parallel_loop(0, CHUNK // 16, unroll=2)
            def _(g):
                bcvec = iota16 + g * 16
                v = plsc.load_gather(rows_v.at[b], [bcvec, hdiag])
                plsc.store_scatter(obuf_v, [sbase + g * 16], v)

    for b in range(LOOKAHEAD):
        gather_start(b, b)

    def body(l, carry):
        b = l % NB
        ob = l % 2
        gather_wait(l, b)

        @pl.when(l >= 2)
        def _():
            out_wait(l - 2, ob)

        transpose_chunk(b, ob)
        out_start(l, ob)
        ln = l + LOOKAHEAD
        bn = ln % NB

        @pl.when(ln < NCHUNK)
        def _():
            gather_start(ln, bn)

        return carry

    lax.fori_loop(0, NCHUNK, body, 0)

    out_wait(NCHUNK - 2, 0)
    out_wait(NCHUNK - 1, 1)


def kernel(conditions, table):
    tail = table[NBLK * W:].reshape(TAIL * H)
    flat = _transpose_kernel(table.T, tail)
    t_lin = flat.reshape(NODES, H)
    idx = conditions.reshape(NW, CHUNK, L).transpose(0, 2, 1)
    out4 = _gather_kernel(idx, t_lin).reshape(L * 4, NW, 8, CHUNK)
    return out4.transpose(1, 3, 0, 2).reshape(B, L * H)
